# baseline (speedup 1.0000x reference)
import jax
import jax.numpy as jnp
from jax.experimental import pallas as pl

_BLOCK_ROWS = 8192


def _zero_block(o_ref):
    o_ref[...] = jnp.zeros_like(o_ref)


def kernel(inputs, gate_w):
    n, d = inputs.shape
    return pl.pallas_call(
        _zero_block,
        grid=(n // _BLOCK_ROWS,),
        out_specs=pl.BlockSpec((_BLOCK_ROWS, d), lambda i: (i, 0)),
        out_shape=jax.ShapeDtypeStruct((n, d), inputs.dtype),
    )()


# speedup vs baseline: 1.0949x; 1.0949x over previous
import jax
import jax.numpy as jnp
from jax.experimental import pallas as pl

_BLOCK_ROWS = 1024


def _zero_block(o_ref):
    o_ref[...] = jnp.zeros_like(o_ref)


def kernel(inputs, gate_w):
    n, d = inputs.shape
    return pl.pallas_call(
        _zero_block,
        grid=(n // _BLOCK_ROWS,),
        out_specs=pl.BlockSpec((_BLOCK_ROWS, d), lambda i: (i, 0)),
        out_shape=jax.ShapeDtypeStruct((n, d), inputs.dtype),
    )()
